# Initial kernel scaffold; baseline (speedup 1.0000x reference)
import jax
import jax.numpy as jnp
from jax.experimental import pallas as pl


def kernel(x, hyperedge_index, W0v, b0v, W0e, b0e, W1v, b1v, W1e, b1e):
    raise NotImplementedError("write your pallas kernel here")


# speedup vs baseline: 3.8865x; 3.8865x over previous
import functools

import jax
import jax.numpy as jnp
from jax import lax
from jax.experimental import pallas as pl
from jax.experimental.pallas import tpu as pltpu
from jax.experimental.pallas import tpu_sc as plsc

N = 10000
D = 128
NNZ = 320000
TILES = 32
CH = 128
PER_TILE = 10240
NCHT = PER_TILE // CH
NNZ_PAD = TILES * PER_TILE
IDX_ROWS = NNZ_PAD // CH
ROWS_PAD = 10240
TRASH = N
RPS = ROWS_PAD // 16

_mesh = plsc.VectorSubcoreMesh(core_axis_name="c", subcore_axis_name="s")


@functools.partial(
    pl.kernel,
    mesh=_mesh,
    out_type=jax.ShapeDtypeStruct((2, ROWS_PAD, D), jnp.float32),
    scratch_types=[
        pltpu.VMEM((NCHT, CH), jnp.int32),
        pltpu.VMEM((NCHT, CH), jnp.int32),
        pltpu.VMEM((CH, D), jnp.float32),
        pltpu.VMEM_SHARED((ROWS_PAD, D), jnp.float32),
        pltpu.SemaphoreType.DMA,
    ],
)
def _spmm(table_hbm, gidx_hbm, sidx_hbm, zrows_hbm, out_hbm,
          gidx_v, sidx_v, rows_v, acc_sh, sem):
    cid = lax.axis_index("c")
    sid = lax.axis_index("s")
    wid = sid * 2 + cid
    pltpu.sync_copy(gidx_hbm.at[pl.ds(wid * NCHT, NCHT)], gidx_v)
    pltpu.sync_copy(sidx_hbm.at[pl.ds(wid * NCHT, NCHT)], sidx_v)
    pltpu.sync_copy(zrows_hbm, acc_sh.at[pl.ds(sid * RPS, RPS)])
    plsc.subcore_barrier()

    def body(j, carry):
        pltpu.async_copy(table_hbm.at[gidx_v.at[j]], rows_v, sem).wait()
        pltpu.sync_copy(rows_v, acc_sh.at[sidx_v.at[j]], add=True)
        return carry

    lax.fori_loop(0, NCHT, body, 0)
    plsc.subcore_barrier()
    pltpu.sync_copy(acc_sh.at[pl.ds(sid * RPS, RPS)],
                    out_hbm.at[cid, pl.ds(sid * RPS, RPS)])


@functools.partial(
    pl.kernel,
    mesh=_mesh,
    out_type=(jax.ShapeDtypeStruct((2, ROWS_PAD, 16), jnp.float32),
              jax.ShapeDtypeStruct((2, ROWS_PAD, 16), jnp.float32)),
    scratch_types=[
        pltpu.VMEM((NCHT, CH), jnp.int32),
        pltpu.VMEM((NCHT, CH), jnp.int32),
        pltpu.VMEM((CH, 16), jnp.float32),
        pltpu.VMEM_SHARED((ROWS_PAD, 16), jnp.float32),
        pltpu.VMEM_SHARED((ROWS_PAD, 16), jnp.float32),
    ],
    compiler_params=pltpu.CompilerParams(use_tc_tiling_on_sc=False),
)
def _degrees(srcs_hbm, eids_hbm, ones_hbm, z16_hbm, dv_out, de_out,
             srcs_v, eids_v, ones_v, accv_sh, acce_sh):
    cid = lax.axis_index("c")
    sid = lax.axis_index("s")
    wid = sid * 2 + cid
    pltpu.sync_copy(srcs_hbm.at[pl.ds(wid * NCHT, NCHT)], srcs_v)
    pltpu.sync_copy(eids_hbm.at[pl.ds(wid * NCHT, NCHT)], eids_v)
    pltpu.sync_copy(ones_hbm, ones_v)
    pltpu.sync_copy(z16_hbm, accv_sh.at[pl.ds(sid * RPS, RPS)])
    pltpu.sync_copy(z16_hbm, acce_sh.at[pl.ds(sid * RPS, RPS)])
    plsc.subcore_barrier()

    def body(j, carry):
        pltpu.sync_copy(ones_v, accv_sh.at[srcs_v.at[j]], add=True)
        pltpu.sync_copy(ones_v, acce_sh.at[eids_v.at[j]], add=True)
        return carry

    lax.fori_loop(0, NCHT, body, 0)
    plsc.subcore_barrier()
    pltpu.sync_copy(accv_sh.at[pl.ds(sid * RPS, RPS)],
                    dv_out.at[cid, pl.ds(sid * RPS, RPS)])
    pltpu.sync_copy(acce_sh.at[pl.ds(sid * RPS, RPS)],
                    de_out.at[cid, pl.ds(sid * RPS, RPS)])


@functools.partial(
    pl.kernel,
    mesh=_mesh,
    out_type=(jax.ShapeDtypeStruct((2, ROWS_PAD, 16), jnp.float32),
              jax.ShapeDtypeStruct((2, ROWS_PAD, 16), jnp.float32)),
    scratch_types=[
        pltpu.VMEM((NCHT, CH), jnp.int32),
        pltpu.VMEM((NCHT, CH), jnp.int32),
        pltpu.VMEM((NCHT, CH), jnp.int32),
        pltpu.VMEM((NCHT, CH), jnp.int32),
        pltpu.VMEM((CH, 16), jnp.float32),
        pltpu.VMEM_SHARED((ROWS_PAD, 16), jnp.float32),
        pltpu.VMEM_SHARED((ROWS_PAD, 16), jnp.float32),
        pltpu.SemaphoreType.DMA,
    ],
    compiler_params=pltpu.CompilerParams(use_tc_tiling_on_sc=False),
)
def _wsums(tdv_hbm, tde_hbm, srcg_hbm, srcs_hbm, eidg_hbm, eids_hbm, z16_hbm,
           en_out, vn_out, srcg_v, srcs_v, eidg_v, eids_v, rows_v,
           accv_sh, acce_sh, sem):
    cid = lax.axis_index("c")
    sid = lax.axis_index("s")
    wid = sid * 2 + cid
    pltpu.sync_copy(srcg_hbm.at[pl.ds(wid * NCHT, NCHT)], srcg_v)
    pltpu.sync_copy(srcs_hbm.at[pl.ds(wid * NCHT, NCHT)], srcs_v)
    pltpu.sync_copy(eidg_hbm.at[pl.ds(wid * NCHT, NCHT)], eidg_v)
    pltpu.sync_copy(eids_hbm.at[pl.ds(wid * NCHT, NCHT)], eids_v)
    pltpu.sync_copy(z16_hbm, accv_sh.at[pl.ds(sid * RPS, RPS)])
    pltpu.sync_copy(z16_hbm, acce_sh.at[pl.ds(sid * RPS, RPS)])
    plsc.subcore_barrier()

    def body(j, carry):
        pltpu.async_copy(tdv_hbm.at[srcg_v.at[j]], rows_v, sem).wait()
        pltpu.sync_copy(rows_v, acce_sh.at[eids_v.at[j]], add=True)
        pltpu.async_copy(tde_hbm.at[eidg_v.at[j]], rows_v, sem).wait()
        pltpu.sync_copy(rows_v, accv_sh.at[srcs_v.at[j]], add=True)
        return carry

    lax.fori_loop(0, NCHT, body, 0)
    plsc.subcore_barrier()
    pltpu.sync_copy(acce_sh.at[pl.ds(sid * RPS, RPS)],
                    en_out.at[cid, pl.ds(sid * RPS, RPS)])
    pltpu.sync_copy(accv_sh.at[pl.ds(sid * RPS, RPS)],
                    vn_out.at[cid, pl.ds(sid * RPS, RPS)])


def _deg_body(dvp_ref, dep_ref, tdv_ref, tde_ref, dvb_ref, dea_ref):
    d_v = jnp.maximum(dvp_ref[0, :, 0:1] + dvp_ref[1, :, 0:1], 1.0)
    d_e = jnp.maximum(dep_ref[0, :, 0:1] + dep_ref[1, :, 0:1], 1.0)
    dv_beta = lax.rsqrt(d_v)
    de_alpha = lax.rsqrt(d_e) / d_e
    dvb_ref[...] = dv_beta
    dea_ref[...] = de_alpha
    tdv_ref[...] = jnp.broadcast_to(dv_beta, (ROWS_PAD, 16))
    tde_ref[...] = jnp.broadcast_to(de_alpha, (ROWS_PAD, 16))


def _deg_tc(dv_p, de_p):
    return pl.pallas_call(
        _deg_body,
        out_shape=(jax.ShapeDtypeStruct((ROWS_PAD, 16), jnp.float32),
                   jax.ShapeDtypeStruct((ROWS_PAD, 16), jnp.float32),
                   jax.ShapeDtypeStruct((ROWS_PAD, 1), jnp.float32),
                   jax.ShapeDtypeStruct((ROWS_PAD, 1), jnp.float32)),
    )(dv_p, de_p)


def _inv_body(enp_ref, vnp_ref, ei_ref, vi_ref):
    en = enp_ref[0, :, 0:1] + enp_ref[1, :, 0:1]
    vn = vnp_ref[0, :, 0:1] + vnp_ref[1, :, 0:1]
    ei_ref[...] = 1.0 / jnp.maximum(en, 1e-12)
    vi_ref[...] = 1.0 / jnp.maximum(vn, 1e-12)


def _inv_tc(en_p, vn_p):
    return pl.pallas_call(
        _inv_body,
        out_shape=(jax.ShapeDtypeStruct((ROWS_PAD, 1), jnp.float32),
                   jax.ShapeDtypeStruct((ROWS_PAD, 1), jnp.float32)),
    )(en_p, vn_p)


def _mm_body(p_ref, spre_ref, w_ref, b_ref, spost_ref, o_ref, *, relu_pre):
    v = jnp.sum(p_ref[...], axis=0) * spre_ref[...]
    if relu_pre:
        v = jnp.maximum(v, 0.0)
    h = jnp.dot(v, w_ref[...], preferred_element_type=jnp.float32) + b_ref[...]
    o_ref[...] = h * spost_ref[...]


_MM_R = 1000


def _mm(p, spre, w, b, spost, relu_pre):
    np_ = p.shape[0]
    return pl.pallas_call(
        functools.partial(_mm_body, relu_pre=relu_pre),
        grid=(N // _MM_R,),
        in_specs=[
            pl.BlockSpec((np_, _MM_R, D), lambda i: (0, i, 0)),
            pl.BlockSpec((_MM_R, 1), lambda i: (i, 0)),
            pl.BlockSpec((D, D), lambda i: (0, 0)),
            pl.BlockSpec((1, D), lambda i: (0, 0)),
            pl.BlockSpec((_MM_R, 1), lambda i: (i, 0)),
        ],
        out_specs=pl.BlockSpec((_MM_R, D), lambda i: (i, 0)),
        out_shape=jax.ShapeDtypeStruct((N, D), jnp.float32),
    )(p, spre, w, b, spost)


def _scale_body(p_ref, s_ref, o_ref):
    o_ref[...] = jnp.sum(p_ref[...], axis=0) * s_ref[...]


def _scale_out(p, s):
    return pl.pallas_call(
        _scale_body,
        grid=(N // _MM_R,),
        in_specs=[
            pl.BlockSpec((2, _MM_R, D), lambda i: (0, i, 0)),
            pl.BlockSpec((_MM_R, 1), lambda i: (i, 0)),
        ],
        out_specs=pl.BlockSpec((_MM_R, D), lambda i: (i, 0)),
        out_shape=jax.ShapeDtypeStruct((N, D), jnp.float32),
    )(p, s)


def kernel(x, hyperedge_index, W0v, b0v, W0e, b0e, W1v, b1v, W1e, b1e):
    src = hyperedge_index[0].astype(jnp.int32)
    eid = hyperedge_index[1].astype(jnp.int32)
    padn = NNZ_PAD - NNZ
    pad0 = jnp.zeros((padn,), jnp.int32)
    padt = jnp.full((padn,), TRASH, jnp.int32)
    srcg = jnp.concatenate([src, pad0]).reshape(IDX_ROWS, CH)
    srcs = jnp.concatenate([src, padt]).reshape(IDX_ROWS, CH)
    eidg = jnp.concatenate([eid, pad0]).reshape(IDX_ROWS, CH)
    eids = jnp.concatenate([eid, padt]).reshape(IDX_ROWS, CH)
    zrows = jnp.zeros((RPS, D), jnp.float32)
    z16 = jnp.zeros((RPS, 16), jnp.float32)
    ones16 = jnp.ones((CH, 16), jnp.float32)
    ones_n = jnp.ones((N, 1), jnp.float32)

    dv_p, de_p = _degrees(srcs, eids, ones16, z16)
    tdv, tde, dv_beta, de_alpha = _deg_tc(dv_p, de_p)
    en_p, vn_p = _wsums(tdv, tde, srcg, srcs, eidg, eids, z16)
    e_inv, v_inv = _inv_tc(en_p, vn_p)
    dv_b, de_a = dv_beta[:N], de_alpha[:N]
    e_i, v_i = e_inv[:N], v_inv[:N]

    h0 = _mm(x[None], ones_n, W0v, b0v.reshape(1, D), dv_b, relu_pre=False)
    pe0 = _spmm(h0, srcg, eids, zrows)
    g0 = _mm(pe0, e_i, W0e, b0e.reshape(1, D), de_a, relu_pre=True)
    pv0 = _spmm(g0, eidg, srcs, zrows)
    h1 = _mm(pv0, v_i, W1v, b1v.reshape(1, D), dv_b, relu_pre=True)
    pe1 = _spmm(h1, srcg, eids, zrows)
    g1 = _mm(pe1, e_i, W1e, b1e.reshape(1, D), de_a, relu_pre=True)
    pv1 = _spmm(g1, eidg, srcs, zrows)
    return _scale_out(pv1, v_i)


# speedup vs baseline: 4.0371x; 1.0388x over previous
import functools

import jax
import jax.numpy as jnp
from jax import lax
from jax.experimental import pallas as pl
from jax.experimental.pallas import tpu as pltpu
from jax.experimental.pallas import tpu_sc as plsc

N = 10000
D = 128
NNZ = 320000
TILES = 32
CH = 128
PER_TILE = 10240
NCHT = PER_TILE // CH
NNZ_PAD = TILES * PER_TILE
IDX_ROWS = NNZ_PAD // CH
NCHT0 = 120
NCHT1 = 160 - NCHT0
NCHT_MAX = max(NCHT0, NCHT1)
ROWS_PAD = 10240
TRASH = N
RPS = ROWS_PAD // 16

_mesh = plsc.VectorSubcoreMesh(core_axis_name="c", subcore_axis_name="s")


@functools.partial(
    pl.kernel,
    mesh=_mesh,
    out_type=jax.ShapeDtypeStruct((2, ROWS_PAD, D), jnp.float32),
    scratch_types=[
        pltpu.VMEM((NCHT_MAX, CH), jnp.int32),
        pltpu.VMEM((NCHT_MAX, CH), jnp.int32),
        pltpu.VMEM((CH, D), jnp.float32),
        pltpu.VMEM_SHARED((ROWS_PAD, D), jnp.float32),
        pltpu.SemaphoreType.DMA,
    ],
)
def _spmm(table_hbm, gidx_hbm, sidx_hbm, zrows_hbm, out_hbm,
          gidx_v, sidx_v, rows_v, acc_sh, sem):
    cid = lax.axis_index("c")
    sid = lax.axis_index("s")
    ncht = lax.select(cid == 0, NCHT0, NCHT1)
    base = cid * 16 * NCHT0 + sid * ncht
    pltpu.sync_copy(gidx_hbm.at[pl.ds(base, NCHT_MAX)], gidx_v)
    pltpu.sync_copy(sidx_hbm.at[pl.ds(base, NCHT_MAX)], sidx_v)
    pltpu.sync_copy(zrows_hbm, acc_sh.at[pl.ds(sid * RPS, RPS)])
    plsc.subcore_barrier()

    def body(j, carry):
        pltpu.async_copy(table_hbm.at[gidx_v.at[j]], rows_v, sem).wait()
        pltpu.sync_copy(rows_v, acc_sh.at[sidx_v.at[j]], add=True)
        return carry

    lax.fori_loop(0, ncht, body, 0)
    plsc.subcore_barrier()
    pltpu.sync_copy(acc_sh.at[pl.ds(sid * RPS, RPS)],
                    out_hbm.at[cid, pl.ds(sid * RPS, RPS)])


@functools.partial(
    pl.kernel,
    mesh=_mesh,
    out_type=(jax.ShapeDtypeStruct((2, ROWS_PAD, 16), jnp.float32),
              jax.ShapeDtypeStruct((2, ROWS_PAD, 16), jnp.float32)),
    scratch_types=[
        pltpu.VMEM((NCHT, CH), jnp.int32),
        pltpu.VMEM((NCHT, CH), jnp.int32),
        pltpu.VMEM((CH, 16), jnp.float32),
        pltpu.VMEM_SHARED((ROWS_PAD, 16), jnp.float32),
        pltpu.VMEM_SHARED((ROWS_PAD, 16), jnp.float32),
    ],
    compiler_params=pltpu.CompilerParams(use_tc_tiling_on_sc=False),
)
def _degrees(srcs_hbm, eids_hbm, ones_hbm, z16_hbm, dv_out, de_out,
             srcs_v, eids_v, ones_v, accv_sh, acce_sh):
    cid = lax.axis_index("c")
    sid = lax.axis_index("s")
    wid = sid * 2 + cid
    pltpu.sync_copy(srcs_hbm.at[pl.ds(wid * NCHT, NCHT)], srcs_v)
    pltpu.sync_copy(eids_hbm.at[pl.ds(wid * NCHT, NCHT)], eids_v)
    pltpu.sync_copy(ones_hbm, ones_v)
    pltpu.sync_copy(z16_hbm, accv_sh.at[pl.ds(sid * RPS, RPS)])
    pltpu.sync_copy(z16_hbm, acce_sh.at[pl.ds(sid * RPS, RPS)])
    plsc.subcore_barrier()

    def body(j, carry):
        pltpu.sync_copy(ones_v, accv_sh.at[srcs_v.at[j]], add=True)
        pltpu.sync_copy(ones_v, acce_sh.at[eids_v.at[j]], add=True)
        return carry

    lax.fori_loop(0, NCHT, body, 0)
    plsc.subcore_barrier()
    pltpu.sync_copy(accv_sh.at[pl.ds(sid * RPS, RPS)],
                    dv_out.at[cid, pl.ds(sid * RPS, RPS)])
    pltpu.sync_copy(acce_sh.at[pl.ds(sid * RPS, RPS)],
                    de_out.at[cid, pl.ds(sid * RPS, RPS)])


@functools.partial(
    pl.kernel,
    mesh=_mesh,
    out_type=(jax.ShapeDtypeStruct((2, ROWS_PAD, 16), jnp.float32),
              jax.ShapeDtypeStruct((2, ROWS_PAD, 16), jnp.float32)),
    scratch_types=[
        pltpu.VMEM((NCHT, CH), jnp.int32),
        pltpu.VMEM((NCHT, CH), jnp.int32),
        pltpu.VMEM((NCHT, CH), jnp.int32),
        pltpu.VMEM((NCHT, CH), jnp.int32),
        pltpu.VMEM((CH, 16), jnp.float32),
        pltpu.VMEM_SHARED((ROWS_PAD, 16), jnp.float32),
        pltpu.VMEM_SHARED((ROWS_PAD, 16), jnp.float32),
        pltpu.SemaphoreType.DMA,
    ],
    compiler_params=pltpu.CompilerParams(use_tc_tiling_on_sc=False),
)
def _wsums(tdv_hbm, tde_hbm, srcg_hbm, srcs_hbm, eidg_hbm, eids_hbm, z16_hbm,
           en_out, vn_out, srcg_v, srcs_v, eidg_v, eids_v, rows_v,
           accv_sh, acce_sh, sem):
    cid = lax.axis_index("c")
    sid = lax.axis_index("s")
    wid = sid * 2 + cid
    pltpu.sync_copy(srcg_hbm.at[pl.ds(wid * NCHT, NCHT)], srcg_v)
    pltpu.sync_copy(srcs_hbm.at[pl.ds(wid * NCHT, NCHT)], srcs_v)
    pltpu.sync_copy(eidg_hbm.at[pl.ds(wid * NCHT, NCHT)], eidg_v)
    pltpu.sync_copy(eids_hbm.at[pl.ds(wid * NCHT, NCHT)], eids_v)
    pltpu.sync_copy(z16_hbm, accv_sh.at[pl.ds(sid * RPS, RPS)])
    pltpu.sync_copy(z16_hbm, acce_sh.at[pl.ds(sid * RPS, RPS)])
    plsc.subcore_barrier()

    def body(j, carry):
        pltpu.async_copy(tdv_hbm.at[srcg_v.at[j]], rows_v, sem).wait()
        pltpu.sync_copy(rows_v, acce_sh.at[eids_v.at[j]], add=True)
        pltpu.async_copy(tde_hbm.at[eidg_v.at[j]], rows_v, sem).wait()
        pltpu.sync_copy(rows_v, accv_sh.at[srcs_v.at[j]], add=True)
        return carry

    lax.fori_loop(0, NCHT, body, 0)
    plsc.subcore_barrier()
    pltpu.sync_copy(acce_sh.at[pl.ds(sid * RPS, RPS)],
                    en_out.at[cid, pl.ds(sid * RPS, RPS)])
    pltpu.sync_copy(accv_sh.at[pl.ds(sid * RPS, RPS)],
                    vn_out.at[cid, pl.ds(sid * RPS, RPS)])


def _deg_body(dvp_ref, dep_ref, tdv_ref, tde_ref, dvb_ref, dea_ref):
    d_v = jnp.maximum(dvp_ref[0, :, 0:1] + dvp_ref[1, :, 0:1], 1.0)
    d_e = jnp.maximum(dep_ref[0, :, 0:1] + dep_ref[1, :, 0:1], 1.0)
    dv_beta = lax.rsqrt(d_v)
    de_alpha = lax.rsqrt(d_e) / d_e
    dvb_ref[...] = dv_beta
    dea_ref[...] = de_alpha
    tdv_ref[...] = jnp.broadcast_to(dv_beta, (ROWS_PAD, 16))
    tde_ref[...] = jnp.broadcast_to(de_alpha, (ROWS_PAD, 16))


def _deg_tc(dv_p, de_p):
    return pl.pallas_call(
        _deg_body,
        out_shape=(jax.ShapeDtypeStruct((ROWS_PAD, 16), jnp.float32),
                   jax.ShapeDtypeStruct((ROWS_PAD, 16), jnp.float32),
                   jax.ShapeDtypeStruct((ROWS_PAD, 1), jnp.float32),
                   jax.ShapeDtypeStruct((ROWS_PAD, 1), jnp.float32)),
    )(dv_p, de_p)


def _inv_body(enp_ref, vnp_ref, ei_ref, vi_ref):
    en = enp_ref[0, :, 0:1] + enp_ref[1, :, 0:1]
    vn = vnp_ref[0, :, 0:1] + vnp_ref[1, :, 0:1]
    ei_ref[...] = 1.0 / jnp.maximum(en, 1e-12)
    vi_ref[...] = 1.0 / jnp.maximum(vn, 1e-12)


def _inv_tc(en_p, vn_p):
    return pl.pallas_call(
        _inv_body,
        out_shape=(jax.ShapeDtypeStruct((ROWS_PAD, 1), jnp.float32),
                   jax.ShapeDtypeStruct((ROWS_PAD, 1), jnp.float32)),
    )(en_p, vn_p)


def _mm_body(p_ref, spre_ref, w_ref, b_ref, spost_ref, o_ref, *, relu_pre):
    v = jnp.sum(p_ref[...], axis=0) * spre_ref[...]
    if relu_pre:
        v = jnp.maximum(v, 0.0)
    h = jnp.dot(v, w_ref[...], preferred_element_type=jnp.float32) + b_ref[...]
    o_ref[...] = h * spost_ref[...]


_MM_R = 1000


def _mm(p, spre, w, b, spost, relu_pre):
    np_ = p.shape[0]
    return pl.pallas_call(
        functools.partial(_mm_body, relu_pre=relu_pre),
        grid=(N // _MM_R,),
        in_specs=[
            pl.BlockSpec((np_, _MM_R, D), lambda i: (0, i, 0)),
            pl.BlockSpec((_MM_R, 1), lambda i: (i, 0)),
            pl.BlockSpec((D, D), lambda i: (0, 0)),
            pl.BlockSpec((1, D), lambda i: (0, 0)),
            pl.BlockSpec((_MM_R, 1), lambda i: (i, 0)),
        ],
        out_specs=pl.BlockSpec((_MM_R, D), lambda i: (i, 0)),
        out_shape=jax.ShapeDtypeStruct((N, D), jnp.float32),
    )(p, spre, w, b, spost)


def _scale_body(p_ref, s_ref, o_ref):
    o_ref[...] = jnp.sum(p_ref[...], axis=0) * s_ref[...]


def _scale_out(p, s):
    return pl.pallas_call(
        _scale_body,
        grid=(N // _MM_R,),
        in_specs=[
            pl.BlockSpec((2, _MM_R, D), lambda i: (0, i, 0)),
            pl.BlockSpec((_MM_R, 1), lambda i: (i, 0)),
        ],
        out_specs=pl.BlockSpec((_MM_R, D), lambda i: (i, 0)),
        out_shape=jax.ShapeDtypeStruct((N, D), jnp.float32),
    )(p, s)


def kernel(x, hyperedge_index, W0v, b0v, W0e, b0e, W1v, b1v, W1e, b1e):
    src = hyperedge_index[0].astype(jnp.int32)
    eid = hyperedge_index[1].astype(jnp.int32)
    padn = NNZ_PAD - NNZ
    pad0 = jnp.zeros((padn,), jnp.int32)
    padt = jnp.full((padn,), TRASH, jnp.int32)
    slack = jnp.zeros((NCHT_MAX * CH,), jnp.int32)
    srcg = jnp.concatenate([src, pad0, slack]).reshape(-1, CH)
    srcs = jnp.concatenate([src, padt, slack]).reshape(-1, CH)
    eidg = jnp.concatenate([eid, pad0, slack]).reshape(-1, CH)
    eids = jnp.concatenate([eid, padt, slack]).reshape(-1, CH)
    zrows = jnp.zeros((RPS, D), jnp.float32)
    z16 = jnp.zeros((RPS, 16), jnp.float32)
    ones16 = jnp.ones((CH, 16), jnp.float32)
    ones_n = jnp.ones((N, 1), jnp.float32)

    dv_p, de_p = _degrees(srcs, eids, ones16, z16)
    tdv, tde, dv_beta, de_alpha = _deg_tc(dv_p, de_p)
    en_p, vn_p = _wsums(tdv, tde, srcg, srcs, eidg, eids, z16)
    e_inv, v_inv = _inv_tc(en_p, vn_p)
    dv_b, de_a = dv_beta[:N], de_alpha[:N]
    e_i, v_i = e_inv[:N], v_inv[:N]

    h0 = _mm(x[None], ones_n, W0v, b0v.reshape(1, D), dv_b, relu_pre=False)
    pe0 = _spmm(h0, srcg, eids, zrows)
    g0 = _mm(pe0, e_i, W0e, b0e.reshape(1, D), de_a, relu_pre=True)
    pv0 = _spmm(g0, eidg, srcs, zrows)
    h1 = _mm(pv0, v_i, W1v, b1v.reshape(1, D), dv_b, relu_pre=True)
    pe1 = _spmm(h1, srcg, eids, zrows)
    g1 = _mm(pe1, e_i, W1e, b1e.reshape(1, D), de_a, relu_pre=True)
    pv1 = _spmm(g1, eidg, srcs, zrows)
    return _scale_out(pv1, v_i)
